# baseline (speedup 1.0000x reference)
import functools

import jax
import jax.numpy as jnp
from jax import lax
from jax.experimental import pallas as pl
from jax.experimental.pallas import tpu as pltpu
from jax.experimental.pallas import tpu_sc as plsc

_NC = 1
_NS = 16
_NW = _NC * _NS
_CHUNK = 128


def _make_sc_agg(b_pad, k, d):
    b_per_w = b_pad // _NW
    chunks = b_per_w // _CHUNK
    assert chunks % 3 == 0 and chunks >= 6
    mesh = plsc.VectorSubcoreMesh(core_axis_name="c", subcore_axis_name="s", num_cores=_NC)

    @functools.partial(
        pl.kernel,
        mesh=mesh,
        out_type=jax.ShapeDtypeStruct((b_pad, d), jnp.float32),
        scratch_types=[
            pltpu.VMEM((k, b_per_w), jnp.int32),
            pltpu.VMEM((_CHUNK, d), jnp.float32),
            pltpu.VMEM((_CHUNK, d), jnp.float32),
            pltpu.VMEM((_CHUNK, d), jnp.float32),
            pltpu.SemaphoreType.DMA,
            pltpu.SemaphoreType.DMA,
            pltpu.SemaphoreType.DMA,
            pltpu.SemaphoreType.DMA,
            pltpu.SemaphoreType.DMA,
            pltpu.SemaphoreType.DMA,
        ],
    )
    def agg(nbrT_hbm, table_hbm, out_hbm, idx_v, acc_a, acc_b, acc_c,
            sem_a, sem_b, sem_c, sem_oa, sem_ob, sem_oc):
        wid = lax.axis_index("c") * _NS + lax.axis_index("s")
        r0 = wid * b_per_w
        pltpu.sync_copy(nbrT_hbm.at[:, pl.ds(r0, b_per_w)], idx_v)

        def zero(acc):
            def zbody(cc, carry):
                for dd in range(d // 16):
                    acc[cc, pl.ds(dd * 16, 16)] = jnp.zeros((16,), jnp.float32)
                return carry

            lax.fori_loop(0, _CHUNK, zbody, 0)

        def issue(j, acc, sem):
            for kk in range(k):
                pltpu.async_copy(
                    table_hbm.at[idx_v.at[kk, pl.ds(j * _CHUNK, _CHUNK)]],
                    acc, sem, add=True,
                )

        def drain_gathers(acc, sem):
            for _ in range(k):
                pltpu.make_async_copy(table_hbm.at[pl.ds(0, _CHUNK)], acc, sem).wait()

        def drain_out(acc, sem_o):
            pltpu.make_async_copy(acc, out_hbm.at[pl.ds(r0, _CHUNK)], sem_o).wait()

        def prep_issue(j, acc, sem, sem_o, first):
            if not first:
                drain_out(acc, sem_o)
            zero(acc)
            issue(j, acc, sem)

        def prep_issue_guarded(j, t, acc, sem, sem_o):
            @pl.when(t > 0)
            def _():
                drain_out(acc, sem_o)

            zero(acc)
            issue(j, acc, sem)

        def complete(j, acc, sem, sem_o):
            drain_gathers(acc, sem)
            pltpu.async_copy(acc, out_hbm.at[pl.ds(r0 + j * _CHUNK, _CHUNK)], sem_o)

        prep_issue(0, acc_a, sem_a, sem_oa, True)
        prep_issue(1, acc_b, sem_b, sem_ob, True)

        def body(t, carry):
            j0 = 3 * t
            prep_issue_guarded(j0 + 2, t, acc_c, sem_c, sem_oc)
            complete(j0, acc_a, sem_a, sem_oa)
            prep_issue(j0 + 3, acc_a, sem_a, sem_oa, False)
            complete(j0 + 1, acc_b, sem_b, sem_ob)
            prep_issue(j0 + 4, acc_b, sem_b, sem_ob, False)
            complete(j0 + 2, acc_c, sem_c, sem_oc)
            return carry

        lax.fori_loop(0, chunks // 3 - 1, body, 0)
        jl = chunks - 3
        prep_issue(jl + 2, acc_c, sem_c, sem_oc, False)
        complete(jl, acc_a, sem_a, sem_oa)
        complete(jl + 1, acc_b, sem_b, sem_ob)
        complete(jl + 2, acc_c, sem_c, sem_oc)
        drain_out(acc_a, sem_oa)
        drain_out(acc_b, sem_ob)
        drain_out(acc_c, sem_oc)

    return agg


def _combine_body(x_ref, a_ref, wm_ref, wn_ref, b_ref, o_ref):
    t = jnp.dot(x_ref[...], wm_ref[...], preferred_element_type=jnp.float32)
    t = t + jnp.dot(a_ref[...], wn_ref[...], preferred_element_type=jnp.float32)
    o_ref[...] = jnp.maximum(t + b_ref[...], 0.0)


def _tc_combine(raw, agg_pad, w_msg, w_nbr, bias):
    m, d_raw = raw.shape
    d_msg = w_msg.shape[1]
    bm = 1024
    grid = (pl.cdiv(m, bm),)
    return pl.pallas_call(
        _combine_body,
        grid=grid,
        in_specs=[
            pl.BlockSpec((bm, d_raw), lambda i: (i, 0)),
            pl.BlockSpec((bm, agg_pad.shape[1]), lambda i: (i, 0)),
            pl.BlockSpec(w_msg.shape, lambda i: (0, 0)),
            pl.BlockSpec(w_nbr.shape, lambda i: (0, 0)),
            pl.BlockSpec(bias.shape, lambda i: (0, 0)),
        ],
        out_specs=pl.BlockSpec((bm, d_msg), lambda i: (i, 0)),
        out_shape=jax.ShapeDtypeStruct((m, d_msg), jnp.float32),
    )(raw, agg_pad, w_msg, w_nbr, bias)


def kernel(raw_messages, neighbors, memory_table, W_msg, b_msg, W_nbr, b_nbr):
    b, k = neighbors.shape
    d = memory_table.shape[1]
    per_w = 3 * _CHUNK * _NW
    n_units = (b + per_w - 1) // per_w
    b_per_w = n_units * 3 * _CHUNK
    b_pad = _NW * b_per_w

    nbrT = jnp.pad(neighbors, ((0, b_pad - b), (0, 0))).T
    agg_pad = _make_sc_agg(b_pad, k, d)(nbrT, memory_table)
    bias = (b_msg + b_nbr).reshape(1, -1)
    return _tc_combine(raw_messages, agg_pad, W_msg, W_nbr, bias)


# speedup vs baseline: 1.1087x; 1.1087x over previous
import functools

import jax
import jax.numpy as jnp
from jax import lax
from jax.experimental import pallas as pl
from jax.experimental.pallas import tpu as pltpu
from jax.experimental.pallas import tpu_sc as plsc

_NC = 1
_NS = 16
_NW = _NC * _NS
_C = 16
_IDX_PER_GATHER = 128


def _make_sc_agg(b_pad, k, d):
    b_per_w = b_pad // _NW
    chunks = b_per_w // _C
    assert chunks % 3 == 0 and chunks >= 6
    idx_n = _C * k
    pieces = [_IDX_PER_GATHER] * (idx_n // _IDX_PER_GATHER)
    if idx_n % _IDX_PER_GATHER:
        pieces.append(idx_n % _IDX_PER_GATHER)
    assert all(p % 8 == 0 for p in pieces)
    mesh = plsc.VectorSubcoreMesh(core_axis_name="c", subcore_axis_name="s", num_cores=_NC)

    @functools.partial(
        pl.kernel,
        mesh=mesh,
        out_type=jax.ShapeDtypeStruct((b_pad, d), jnp.float32),
        compiler_params=pltpu.CompilerParams(
            needs_layout_passes=False, use_tc_tiling_on_sc=False),
        scratch_types=[
            pltpu.VMEM((b_per_w * k,), jnp.int32),
            pltpu.VMEM((idx_n, d // 2), jnp.int32),
            pltpu.VMEM((idx_n, d // 2), jnp.int32),
            pltpu.VMEM((idx_n, d // 2), jnp.int32),
            pltpu.VMEM((_C, d), jnp.float32),
            pltpu.VMEM((_C, d), jnp.float32),
            pltpu.VMEM((_C, d), jnp.float32),
            pltpu.SemaphoreType.DMA,
            pltpu.SemaphoreType.DMA,
            pltpu.SemaphoreType.DMA,
            pltpu.SemaphoreType.DMA,
            pltpu.SemaphoreType.DMA,
            pltpu.SemaphoreType.DMA,
        ],
    )
    def agg(nbr_hbm, table_hbm, out_hbm, idx_v, rows_a, rows_b, rows_c,
            acc_a, acc_b, acc_c, sga, sgb, sgc, soa, sob, soc):
        wid = lax.axis_index("c") * _NS + lax.axis_index("s")
        r0 = wid * b_per_w
        pltpu.sync_copy(nbr_hbm.at[pl.ds(r0 * k, b_per_w * k)], idx_v)

        def issue(j, rows, sem):
            off = j * idx_n
            o = 0
            for p in pieces:
                pltpu.async_copy(
                    table_hbm.at[idx_v.at[pl.ds(off + o, p)]],
                    rows.at[pl.ds(o, p)],
                    sem,
                )
                o += p

        def drain_gathers(rows, sem):
            pltpu.make_async_copy(table_hbm.at[pl.ds(0, idx_n)], rows, sem).wait()

        def drain_out(acc, sem_o):
            pltpu.make_async_copy(acc, out_hbm.at[pl.ds(r0, _C)], sem_o).wait()

        def complete(j, t, rows, sem, acc, sem_o):
            drain_gathers(rows, sem)
            if t is None:
                drain_out(acc, sem_o)
            else:
                @pl.when(t > 0)
                def _():
                    drain_out(acc, sem_o)

            def red_body(cc, carry):
                rbase = cc * k
                for g in range(d // 32):
                    sl = pl.ds(g * 16, 16)
                    w = plsc.bitcast(rows[rbase, sl], jnp.bfloat16)
                    ea, eb = plsc.unpack(w, format=plsc.PackFormat.INTERLEAVED)
                    for kk in range(1, k):
                        w = plsc.bitcast(rows[rbase + kk, sl], jnp.bfloat16)
                        a2, b2 = plsc.unpack(w, format=plsc.PackFormat.INTERLEAVED)
                        ea = ea + a2
                        eb = eb + b2
                    acc[cc, pl.ds(g * 32, 16)] = ea
                    acc[cc, pl.ds(g * 32 + 16, 16)] = eb
                return carry

            lax.fori_loop(0, _C, red_body, 0)
            pltpu.async_copy(acc, out_hbm.at[pl.ds(r0 + j * _C, _C)], sem_o)

        issue(0, rows_a, sga)
        issue(1, rows_b, sgb)

        def body(t, carry):
            j0 = 3 * t
            issue(j0 + 2, rows_c, sgc)
            complete(j0, t, rows_a, sga, acc_a, soa)
            issue(j0 + 3, rows_a, sga)
            complete(j0 + 1, t, rows_b, sgb, acc_b, sob)
            issue(j0 + 4, rows_b, sgb)
            complete(j0 + 2, t, rows_c, sgc, acc_c, soc)
            return carry

        lax.fori_loop(0, chunks // 3 - 1, body, 0)
        jl = chunks - 3
        issue(jl + 2, rows_c, sgc)
        complete(jl, None, rows_a, sga, acc_a, soa)
        complete(jl + 1, None, rows_b, sgb, acc_b, sob)
        complete(jl + 2, None, rows_c, sgc, acc_c, soc)
        drain_out(acc_a, soa)
        drain_out(acc_b, sob)
        drain_out(acc_c, soc)

    return agg


def _combine_body(x_ref, a_ref, wm_ref, wn_ref, b_ref, o_ref):
    t = jnp.dot(x_ref[...], wm_ref[...], preferred_element_type=jnp.float32)
    t = t + jnp.dot(a_ref[...], wn_ref[...], preferred_element_type=jnp.float32)
    o_ref[...] = jnp.maximum(t + b_ref[...], 0.0)


def _tc_combine(raw, agg_pad, w_msg, w_nbr, bias):
    m, d_raw = raw.shape
    d_msg = w_msg.shape[1]
    bm = 1024
    grid = (pl.cdiv(m, bm),)
    return pl.pallas_call(
        _combine_body,
        grid=grid,
        in_specs=[
            pl.BlockSpec((bm, d_raw), lambda i: (i, 0)),
            pl.BlockSpec((bm, agg_pad.shape[1]), lambda i: (i, 0)),
            pl.BlockSpec(w_msg.shape, lambda i: (0, 0)),
            pl.BlockSpec(w_nbr.shape, lambda i: (0, 0)),
            pl.BlockSpec(bias.shape, lambda i: (0, 0)),
        ],
        out_specs=pl.BlockSpec((bm, d_msg), lambda i: (i, 0)),
        out_shape=jax.ShapeDtypeStruct((m, d_msg), jnp.float32),
    )(raw, agg_pad, w_msg, w_nbr, bias)


def kernel(raw_messages, neighbors, memory_table, W_msg, b_msg, W_nbr, b_nbr):
    b, k = neighbors.shape
    d = memory_table.shape[1]
    per_w = 3 * _C * _NW
    n_units = (b + per_w - 1) // per_w
    b_per_w = n_units * 3 * _C
    b_pad = _NW * b_per_w

    nbr_flat = jnp.pad(neighbors.reshape(-1), (0, (b_pad - b) * k))
    table_bf = memory_table.astype(jnp.bfloat16)
    table_w = jax.lax.bitcast_convert_type(
        table_bf.reshape(memory_table.shape[0], d // 2, 2), jnp.int32)
    agg_pad = _make_sc_agg(b_pad, k, d)(nbr_flat, table_w)

    perm = []
    for g in range(d // 32):
        perm.extend(32 * g + 2 * i for i in range(16))
        perm.extend(32 * g + 2 * i + 1 for i in range(16))
    w_nbr_perm = W_nbr[jnp.array(perm, dtype=jnp.int32), :]

    bias = (b_msg + b_nbr).reshape(1, -1)
    return _tc_combine(raw_messages, agg_pad, W_msg, w_nbr_perm, bias)


# speedup vs baseline: 1.9029x; 1.7163x over previous
import functools

import jax
import jax.numpy as jnp
from jax import lax
from jax.experimental import pallas as pl
from jax.experimental.pallas import tpu as pltpu
from jax.experimental.pallas import tpu_sc as plsc

_NC = 2
_NS = 16
_UNIT = 32
_SLOW_CORE = 0
_IDX_PER_GATHER = 128


def _make_sc_agg(b_pad, k, d, c_chunk, u_slow):
    units = b_pad // (_NS * _UNIT)
    u_fast = units - u_slow
    slow_rows = u_slow * _UNIT
    fast_rows = u_fast * _UNIT
    assert (slow_rows // c_chunk) % 2 == 0 and (fast_rows // c_chunk) % 2 == 0
    idx_n = c_chunk * k
    pieces = [_IDX_PER_GATHER] * (idx_n // _IDX_PER_GATHER)
    if idx_n % _IDX_PER_GATHER:
        pieces.append(idx_n % _IDX_PER_GATHER)
    assert all(p % 8 == 0 for p in pieces)
    mesh = plsc.VectorSubcoreMesh(core_axis_name="c", subcore_axis_name="s", num_cores=_NC)

    @functools.partial(
        pl.kernel,
        mesh=mesh,
        out_type=jax.ShapeDtypeStruct((b_pad, d), jnp.float32),
        scratch_types=[
            pltpu.VMEM((fast_rows * k,), jnp.int32),
            pltpu.VMEM((idx_n, d), jnp.float32),
            pltpu.VMEM((idx_n, d), jnp.float32),
            pltpu.VMEM((c_chunk, d), jnp.float32),
            pltpu.VMEM((c_chunk, d), jnp.float32),
            pltpu.SemaphoreType.DMA,
            pltpu.SemaphoreType.DMA,
            pltpu.SemaphoreType.DMA,
        ],
    )
    def agg(nbr_hbm, table_hbm, out_hbm, idx_v, rows_a, rows_b, acc_a, acc_b,
            sem_a, sem_b, sem_o):
        cc_ax = lax.axis_index("c")
        s_ax = lax.axis_index("s")
        is_slow = cc_ax == _SLOW_CORE
        r0 = jnp.where(is_slow, s_ax * slow_rows,
                       _NS * slow_rows + s_ax * fast_rows)
        nch = jnp.where(is_slow, slow_rows // c_chunk, fast_rows // c_chunk)

        @pl.when(is_slow)
        def _():
            pltpu.sync_copy(nbr_hbm.at[pl.ds(r0 * k, slow_rows * k)],
                            idx_v.at[pl.ds(0, slow_rows * k)])

        @pl.when(jnp.logical_not(is_slow))
        def _():
            pltpu.sync_copy(nbr_hbm.at[pl.ds(r0 * k, fast_rows * k)], idx_v)

        def issue(j, rows, sem):
            off = j * idx_n
            o = 0
            for p in pieces:
                pltpu.async_copy(
                    table_hbm.at[idx_v.at[pl.ds(off + o, p)]],
                    rows.at[pl.ds(o, p)],
                    sem,
                )
                o += p

        def drain(rows, sem):
            pltpu.make_async_copy(table_hbm.at[pl.ds(0, idx_n)], rows, sem).wait()

        def reduce_store(j, t, rows, acc):
            @pl.when(t > 0)
            def _():
                pltpu.make_async_copy(acc, out_hbm.at[pl.ds(r0, c_chunk)], sem_o).wait()

            def red_body(cc, carry2):
                rbase = cc * k
                for dd in range(d // 16):
                    sl = pl.ds(dd * 16, 16)
                    s = rows[rbase, sl]
                    for kk in range(1, k):
                        s = s + rows[rbase + kk, sl]
                    acc[cc, sl] = s
                return carry2

            lax.fori_loop(0, c_chunk, red_body, 0)
            pltpu.async_copy(acc, out_hbm.at[pl.ds(r0 + j * c_chunk, c_chunk)], sem_o)

        issue(0, rows_a, sem_a)

        def pair_body(t, carry):
            j0 = 2 * t
            issue(j0 + 1, rows_b, sem_b)
            drain(rows_a, sem_a)
            reduce_store(j0, t, rows_a, acc_a)
            issue(jnp.where(j0 + 2 < nch, j0 + 2, 0), rows_a, sem_a)
            drain(rows_b, sem_b)
            reduce_store(j0 + 1, t, rows_b, acc_b)
            return carry

        lax.fori_loop(0, nch // 2, pair_body, 0)
        drain(rows_a, sem_a)
        pltpu.make_async_copy(acc_a, out_hbm.at[pl.ds(r0, c_chunk)], sem_o).wait()
        pltpu.make_async_copy(acc_b, out_hbm.at[pl.ds(r0, c_chunk)], sem_o).wait()

    return agg


def _combine_body(x_ref, a_ref, wm_ref, wn_ref, b_ref, o_ref):
    t = jnp.dot(x_ref[...], wm_ref[...], preferred_element_type=jnp.float32)
    t = t + jnp.dot(a_ref[...], wn_ref[...], preferred_element_type=jnp.float32)
    o_ref[...] = jnp.maximum(t + b_ref[...], 0.0)


def _tc_combine(raw, agg_pad, w_msg, w_nbr, bias):
    m, d_raw = raw.shape
    d_msg = w_msg.shape[1]
    bm = 1024
    grid = (pl.cdiv(m, bm),)
    return pl.pallas_call(
        _combine_body,
        grid=grid,
        in_specs=[
            pl.BlockSpec((bm, d_raw), lambda i: (i, 0)),
            pl.BlockSpec((bm, agg_pad.shape[1]), lambda i: (i, 0)),
            pl.BlockSpec(w_msg.shape, lambda i: (0, 0)),
            pl.BlockSpec(w_nbr.shape, lambda i: (0, 0)),
            pl.BlockSpec(bias.shape, lambda i: (0, 0)),
        ],
        out_specs=pl.BlockSpec((bm, d_msg), lambda i: (i, 0)),
        out_shape=jax.ShapeDtypeStruct((m, d_msg), jnp.float32),
    )(raw, agg_pad, w_msg, w_nbr, bias)


def kernel(raw_messages, neighbors, memory_table, W_msg, b_msg, W_nbr, b_nbr):
    b, k = neighbors.shape
    d = memory_table.shape[1]
    c_chunk = 16
    unit_rows = _NS * _UNIT
    units = (b + unit_rows - 1) // unit_rows
    b_pad = units * unit_rows
    u_slow = max(2, round(0.2 * units))

    nbr_flat = jnp.pad(neighbors.reshape(-1), (0, (b_pad - b) * k))
    agg_pad = _make_sc_agg(b_pad, k, d, c_chunk, u_slow)(nbr_flat, memory_table)
    bias = (b_msg + b_nbr).reshape(1, -1)
    return _tc_combine(raw_messages, agg_pad, W_msg, W_nbr, bias)
